# baseline (speedup 1.0000x reference)
import functools

import jax
import jax.numpy as jnp
from jax import lax
from jax.experimental import pallas as pl
from jax.experimental.pallas import tpu as pltpu
from jax.experimental.pallas import tpu_sc as plsc

_N = 8192
_K = 8192
_D = 32
_TN = 1024


def _normalize_body(x_ref, embed_ref, xn_ref, en_ref):
    xb = x_ref[...]
    xn_ref[...] = xb / jnp.clip(
        jnp.sqrt(jnp.sum(xb * xb, axis=1, keepdims=True)), 1e-12)
    e = embed_ref[...]
    en_ref[...] = e / jnp.clip(
        jnp.sqrt(jnp.sum(e * e, axis=1, keepdims=True)), 1e-12)


def _normalize(flat_x, embed):
    return pl.pallas_call(
        _normalize_body,
        out_shape=(
            jax.ShapeDtypeStruct((_N, _D), jnp.float32),
            jax.ShapeDtypeStruct((_K, _D), jnp.float32),
        ),
    )(flat_x, embed)


def _argmax_body(xn_ref, en_ref, ind_ref):
    d = lax.dot_general(
        xn_ref[...], en_ref[...],
        (((1,), (1,)), ((), ())),
        preferred_element_type=jnp.float32)
    ind_ref[...] = jnp.argmax(d, axis=1).astype(jnp.int32)


def _argmax_indices(xn, en):
    return pl.pallas_call(
        _argmax_body,
        grid=(_N // _TN,),
        in_specs=[
            pl.BlockSpec((_TN, _D), lambda i: (i, 0)),
            pl.BlockSpec((_K, _D), lambda i: (0, 0)),
        ],
        out_specs=pl.BlockSpec((_TN,), lambda i: (i,)),
        out_shape=jax.ShapeDtypeStruct((_N,), jnp.int32),
        compiler_params=pltpu.CompilerParams(
            dimension_semantics=("parallel",)),
    )(xn, en)


@functools.cache
def _sc_gather_kernel():
    info = plsc.get_sparse_core_info()
    nw = info.num_cores * info.num_subcores
    b_per_w = _N // nw
    mesh = plsc.VectorSubcoreMesh(core_axis_name="c", subcore_axis_name="s")

    @functools.partial(
        pl.kernel,
        out_type=jax.ShapeDtypeStruct((_N, _D), jnp.float32),
        mesh=mesh,
        scratch_types=[
            pltpu.VMEM((b_per_w,), jnp.int32),
            pltpu.VMEM((b_per_w, _D), jnp.float32),
            pltpu.SemaphoreType.DMA,
        ],
        compiler_params=pltpu.CompilerParams(use_tc_tiling_on_sc=False),
    )
    def gather(table_hbm, idx_hbm, out_hbm, idx_v, rows_v, sem):
        wid = lax.axis_index("s") * info.num_cores + lax.axis_index("c")
        base = wid * b_per_w
        pltpu.sync_copy(idx_hbm.at[pl.ds(base, b_per_w)], idx_v)
        pltpu.async_copy(table_hbm.at[idx_v], rows_v, sem).wait()
        pltpu.sync_copy(rows_v, out_hbm.at[pl.ds(base, b_per_w)])

    return gather


def kernel(x, embed):
    shape = x.shape
    flat = x.reshape(-1, shape[-1])
    xn, en = _normalize(flat, embed)
    ind = _argmax_indices(xn, en)
    quantize = _sc_gather_kernel()(embed, ind)
    return (quantize.reshape(shape), ind.reshape(shape[:-1]))


# speedup vs baseline: 1.1073x; 1.1073x over previous
import functools

import jax
import jax.numpy as jnp
from jax import lax
from jax.experimental import pallas as pl
from jax.experimental.pallas import tpu as pltpu
from jax.experimental.pallas import tpu_sc as plsc

_N = 8192
_K = 8192
_D = 32
_TN = 1024
_SN = 512


def _argmax_body(x_ref, embed_ref, ind_ref, en_ref):
    @pl.when(pl.program_id(0) == 0)
    def _():
        e = embed_ref[...]
        en_ref[...] = e / jnp.clip(
            jnp.sqrt(jnp.sum(e * e, axis=1, keepdims=True)), 1e-12)

    for s in range(0, _TN, _SN):
        xb = x_ref[s:s + _SN, :]
        xn = xb / jnp.clip(
            jnp.sqrt(jnp.sum(xb * xb, axis=1, keepdims=True)), 1e-12)
        d = lax.dot_general(
            xn, en_ref[...],
            (((1,), (1,)), ((), ())),
            preferred_element_type=jnp.float32)
        ind_ref[s:s + _SN] = jnp.argmax(d, axis=1).astype(jnp.int32)


def _argmax_indices(flat_x, embed):
    return pl.pallas_call(
        _argmax_body,
        grid=(_N // _TN,),
        in_specs=[
            pl.BlockSpec((_TN, _D), lambda i: (i, 0)),
            pl.BlockSpec((_K, _D), lambda i: (0, 0)),
        ],
        out_specs=pl.BlockSpec((_TN,), lambda i: (i,)),
        out_shape=jax.ShapeDtypeStruct((_N,), jnp.int32),
        scratch_shapes=[pltpu.VMEM((_K, _D), jnp.float32)],
    )(flat_x, embed)


@functools.cache
def _sc_gather_kernel():
    info = plsc.get_sparse_core_info()
    nw = info.num_cores * info.num_subcores
    b_per_w = _N // nw
    mesh = plsc.VectorSubcoreMesh(core_axis_name="c", subcore_axis_name="s")

    @functools.partial(
        pl.kernel,
        out_type=jax.ShapeDtypeStruct((_N, _D), jnp.float32),
        mesh=mesh,
        scratch_types=[
            pltpu.VMEM((b_per_w,), jnp.int32),
            pltpu.VMEM((b_per_w, _D), jnp.float32),
            pltpu.SemaphoreType.DMA,
        ],
        compiler_params=pltpu.CompilerParams(use_tc_tiling_on_sc=False),
    )
    def gather(table_hbm, idx_hbm, out_hbm, idx_v, rows_v, sem):
        wid = lax.axis_index("s") * info.num_cores + lax.axis_index("c")
        base = wid * b_per_w
        pltpu.sync_copy(idx_hbm.at[pl.ds(base, b_per_w)], idx_v)
        pltpu.async_copy(table_hbm.at[idx_v], rows_v, sem).wait()
        pltpu.sync_copy(rows_v, out_hbm.at[pl.ds(base, b_per_w)])

    return gather


def kernel(x, embed):
    shape = x.shape
    flat = x.reshape(-1, shape[-1])
    ind = _argmax_indices(flat, embed)
    quantize = _sc_gather_kernel()(embed, ind)
    return (quantize.reshape(shape), ind.reshape(shape[:-1]))


# speedup vs baseline: 1.1216x; 1.0129x over previous
import functools

import jax
import jax.numpy as jnp
from jax import lax
from jax.experimental import pallas as pl
from jax.experimental.pallas import tpu as pltpu
from jax.experimental.pallas import tpu_sc as plsc

_N = 8192
_K = 8192
_D = 32
_TN = 2048
_SN = 512


def _argmax_body(x_ref, embed_ref, ind_ref, en_ref):
    @pl.when(pl.program_id(0) == 0)
    def _():
        e = embed_ref[...]
        en_ref[...] = e / jnp.clip(
            jnp.sqrt(jnp.sum(e * e, axis=1, keepdims=True)), 1e-12)

    for s in range(0, _TN, _SN):
        xb = x_ref[s:s + _SN, :]
        xn = xb / jnp.clip(
            jnp.sqrt(jnp.sum(xb * xb, axis=1, keepdims=True)), 1e-12)
        d = lax.dot_general(
            xn, en_ref[...],
            (((1,), (1,)), ((), ())),
            preferred_element_type=jnp.float32)
        ind_ref[s:s + _SN] = jnp.argmax(d, axis=1).astype(jnp.int32)


def _argmax_indices(flat_x, embed):
    return pl.pallas_call(
        _argmax_body,
        grid=(_N // _TN,),
        in_specs=[
            pl.BlockSpec((_TN, _D), lambda i: (i, 0)),
            pl.BlockSpec((_K, _D), lambda i: (0, 0)),
        ],
        out_specs=pl.BlockSpec((_TN,), lambda i: (i,)),
        out_shape=jax.ShapeDtypeStruct((_N,), jnp.int32),
        scratch_shapes=[pltpu.VMEM((_K, _D), jnp.float32)],
        compiler_params=pltpu.CompilerParams(
            vmem_limit_bytes=120 * 1024 * 1024),
    )(flat_x, embed)


@functools.cache
def _sc_gather_kernel():
    info = plsc.get_sparse_core_info()
    nw = info.num_cores * info.num_subcores
    b_per_w = _N // nw
    mesh = plsc.VectorSubcoreMesh(core_axis_name="c", subcore_axis_name="s")

    @functools.partial(
        pl.kernel,
        out_type=jax.ShapeDtypeStruct((_N, _D), jnp.float32),
        mesh=mesh,
        scratch_types=[
            pltpu.VMEM((b_per_w,), jnp.int32),
            pltpu.VMEM((b_per_w, _D), jnp.float32),
            pltpu.SemaphoreType.DMA,
        ],
        compiler_params=pltpu.CompilerParams(use_tc_tiling_on_sc=False),
    )
    def gather(table_hbm, idx_hbm, out_hbm, idx_v, rows_v, sem):
        wid = lax.axis_index("s") * info.num_cores + lax.axis_index("c")
        base = wid * b_per_w
        pltpu.sync_copy(idx_hbm.at[pl.ds(base, b_per_w)], idx_v)
        pltpu.async_copy(table_hbm.at[idx_v], rows_v, sem).wait()
        pltpu.sync_copy(rows_v, out_hbm.at[pl.ds(base, b_per_w)])

    return gather


def kernel(x, embed):
    shape = x.shape
    flat = x.reshape(-1, shape[-1])
    ind = _argmax_indices(flat, embed)
    quantize = _sc_gather_kernel()(embed, ind)
    return (quantize.reshape(shape), ind.reshape(shape[:-1]))
